# Initial kernel scaffold; baseline (speedup 1.0000x reference)
import jax
import jax.numpy as jnp
from jax.experimental import pallas as pl


def kernel(gene_set_features):
    raise NotImplementedError("write your pallas kernel here")


# speedup vs baseline: 1.1726x; 1.1726x over previous
import functools

import jax
import jax.numpy as jnp
from jax import lax
from jax.experimental import pallas as pl
from jax.experimental.pallas import tpu as pltpu
from jax.experimental.pallas import tpu_sc as plsc

B = 16384
F = 512
G = 8
P = F // G
L = 16
NC = 2
NS = 16
NW = NC * NS

ROWS_PER_W = B // NW
CH = 64
NCHUNK = ROWS_PER_W // CH
IN_CHUNK = CH * F
OUT_CHUNK = CH * P

_mesh = plsc.VectorSubcoreMesh(core_axis_name="c", subcore_axis_name="s")


@functools.partial(
    pl.kernel,
    out_type=jax.ShapeDtypeStruct((B * P,), jnp.float32),
    mesh=_mesh,
    scratch_types=[
        pltpu.VMEM((IN_CHUNK,), jnp.float32),
        pltpu.VMEM((IN_CHUNK,), jnp.float32),
        pltpu.VMEM((OUT_CHUNK,), jnp.float32),
        pltpu.VMEM((OUT_CHUNK,), jnp.float32),
        pltpu.SemaphoreType.DMA,
        pltpu.SemaphoreType.DMA,
        pltpu.SemaphoreType.DMA,
        pltpu.SemaphoreType.DMA,
    ],
    compiler_params=pltpu.CompilerParams(needs_layout_passes=False),
)
def _pool_sc(x_hbm, out_hbm, in0, in1, o0, o1, si0, si1, so0, so1):
    wid = lax.axis_index("s") * NC + lax.axis_index("c")
    ibase = wid * (ROWS_PER_W * F)
    obase = wid * (ROWS_PER_W * P)

    ins = (in0, in1)
    outs = (o0, o1)
    isems = (si0, si1)
    osems = (so0, so1)

    lane8 = lax.iota(jnp.int32, L) * G

    in_copies = [None, None]
    out_copies = [None, None]
    in_copies[0] = pltpu.async_copy(
        x_hbm.at[pl.ds(ibase, IN_CHUNK)], ins[0], isems[0]
    )

    for c in range(NCHUNK):
        cur = c % 2
        if c + 1 < NCHUNK:
            nxt = (c + 1) % 2
            in_copies[nxt] = pltpu.async_copy(
                x_hbm.at[pl.ds(ibase + (c + 1) * IN_CHUNK, IN_CHUNK)],
                ins[nxt],
                isems[nxt],
            )
        in_copies[cur].wait()
        if out_copies[cur] is not None:
            out_copies[cur].wait()

        in_ref = ins[cur]
        out_ref = outs[cur]

        @plsc.parallel_loop(0, OUT_CHUNK, step=L, unroll=4)
        def _body(i):
            base_idx = lane8 + i * G
            acc = plsc.load_gather(in_ref, [base_idx])
            for k in range(1, G):
                acc = acc + plsc.load_gather(in_ref, [base_idx + k])
            out_ref[pl.ds(i, L)] = acc * (1.0 / G)

        out_copies[cur] = pltpu.async_copy(
            out_ref,
            out_hbm.at[pl.ds(obase + c * OUT_CHUNK, OUT_CHUNK)],
            osems[cur],
        )

    out_copies[0].wait()
    out_copies[1].wait()


def kernel(gene_set_features):
    flat = gene_set_features.reshape(-1)
    out = _pool_sc(flat)
    return out.reshape(B, P)


# speedup vs baseline: 2.0225x; 1.7248x over previous
import functools

import jax
import jax.numpy as jnp
from jax import lax
from jax.experimental import pallas as pl
from jax.experimental.pallas import tpu as pltpu
from jax.experimental.pallas import tpu_sc as plsc

B = 16384
F = 512
G = 8
P = F // G
L = 16
NC = 2
NS = 16
NW = NC * NS

ROWS_PER_W = B // NW
CH = 64
NCHUNK = ROWS_PER_W // CH

_mesh = plsc.VectorSubcoreMesh(core_axis_name="c", subcore_axis_name="s")


@functools.partial(
    pl.kernel,
    out_type=jax.ShapeDtypeStruct((B, P), jnp.float32),
    mesh=_mesh,
    scratch_types=[
        pltpu.VMEM((CH, F), jnp.float32),
        pltpu.VMEM((CH, F), jnp.float32),
        pltpu.VMEM((CH, P), jnp.float32),
        pltpu.VMEM((CH, P), jnp.float32),
        pltpu.SemaphoreType.DMA,
        pltpu.SemaphoreType.DMA,
        pltpu.SemaphoreType.DMA,
        pltpu.SemaphoreType.DMA,
    ],
    compiler_params=pltpu.CompilerParams(needs_layout_passes=False),
)
def _pool_sc(x_hbm, out_hbm, in0, in1, o0, o1, si0, si1, so0, so1):
    wid = lax.axis_index("s") * NC + lax.axis_index("c")
    row0 = wid * ROWS_PER_W

    ins = (in0, in1)
    outs = (o0, o1)
    isems = (si0, si1)
    osems = (so0, so1)

    lane8 = lax.iota(jnp.int32, L) * G

    in_copies = [None, None]
    out_copies = [None, None]
    in_copies[0] = pltpu.async_copy(
        x_hbm.at[pl.ds(row0, CH)], ins[0], isems[0]
    )

    for c in range(NCHUNK):
        cur = c % 2
        if c + 1 < NCHUNK:
            nxt = (c + 1) % 2
            in_copies[nxt] = pltpu.async_copy(
                x_hbm.at[pl.ds(row0 + (c + 1) * CH, CH)],
                ins[nxt],
                isems[nxt],
            )
        in_copies[cur].wait()
        if out_copies[cur] is not None:
            out_copies[cur].wait()

        in_ref = ins[cur]
        out_ref = outs[cur]

        @plsc.parallel_loop(0, CH, step=1, unroll=2)
        def _body(r):
            row_idx = jnp.full((L,), r, dtype=jnp.int32)
            for g in range(P // L):
                col0 = lane8 + g * (L * G)
                acc = plsc.load_gather(in_ref, [row_idx, col0])
                for k in range(1, G):
                    acc = acc + plsc.load_gather(in_ref, [row_idx, col0 + k])
                out_ref[r, pl.ds(g * L, L)] = acc * (1.0 / G)

        out_copies[cur] = pltpu.async_copy(
            out_ref,
            out_hbm.at[pl.ds(row0 + c * CH, CH)],
            osems[cur],
        )

    out_copies[0].wait()
    out_copies[1].wait()


def kernel(gene_set_features):
    return _pool_sc(gene_set_features)
